# Initial kernel scaffold; baseline (speedup 1.0000x reference)
import jax
import jax.numpy as jnp
from jax.experimental import pallas as pl


def kernel(e_prev, edge_index, W1, b1, W2, b2):
    raise NotImplementedError("write your pallas kernel here")


# speedup vs baseline: 12.0014x; 12.0014x over previous
import functools

import jax
import jax.numpy as jnp
from jax import lax
from jax.experimental import pallas as pl
from jax.experimental.pallas import tpu as pltpu
from jax.experimental.pallas import tpu_sc as plsc

NC = 2
NS = 16
NW = NC * NS
LW = 128
PAD_ROWS = 16
ROWS_TC = 2000


def _sc_mesh():
    return plsc.VectorSubcoreMesh(core_axis_name="c", subcore_axis_name="s")


CH = 624


def _sliced_copy(s, total, src_at, dst_at):
    last = total - (NS - 1) * CH
    off = pl.multiple_of(s * CH, 8)

    @pl.when(s < NS - 1)
    def _():
        pltpu.sync_copy(src_at(off, CH), dst_at(off, CH))

    @pl.when(s == NS - 1)
    def _():
        o = (NS - 1) * CH
        pltpu.sync_copy(src_at(o, last), dst_at(o, last))


def _deg_call(n, n_pad, k, d, dstr, zeros_nd, ones_rows):
    @functools.partial(
        pl.kernel,
        mesh=_sc_mesh(),
        out_type=jax.ShapeDtypeStruct((NC, n, d), jnp.float32),
        scratch_types=[
            pltpu.VMEM((k, LW), jnp.int32),
            pltpu.VMEM((LW, d), jnp.float32),
            pltpu.VMEM_SHARED((n_pad, d), jnp.float32),
            pltpu.SemaphoreType.DMA,
        ],
    )
    def deg_kernel(dst_hbm, zeros_hbm, ones_hbm, out_hbm, didx, ones_v, acc, sem):
        c = lax.axis_index("c")
        s = lax.axis_index("s")
        wid = c * NS + s
        _sliced_copy(s, n,
                     lambda o, r: zeros_hbm.at[pl.ds(o, r)],
                     lambda o, r: acc.at[pl.ds(o, r)])
        pltpu.sync_copy(ones_hbm, ones_v)
        pltpu.sync_copy(dst_hbm.at[wid], didx)
        plsc.subcore_barrier()

        @pl.loop(0, k)
        def _(j):
            pltpu.sync_copy(ones_v, acc.at[didx.at[j]], add=True)

        plsc.subcore_barrier()
        _sliced_copy(s, n,
                     lambda o, r: acc.at[pl.ds(o, r)],
                     lambda o, r: out_hbm.at[c].at[pl.ds(o, r)])

    return deg_kernel(dstr, zeros_nd, ones_rows)


def _agg_call(n, n_pad, k, d, hp, zeros_nd, srcr, dstr):
    @functools.partial(
        pl.kernel,
        mesh=_sc_mesh(),
        out_type=jax.ShapeDtypeStruct((NC, n, d), jnp.float32),
        scratch_types=[
            pltpu.VMEM((k, LW), jnp.int32),
            pltpu.VMEM((k, LW), jnp.int32),
            pltpu.VMEM((LW, d), jnp.float32),
            pltpu.VMEM_SHARED((n_pad, d), jnp.float32),
            pltpu.SemaphoreType.DMA,
        ],
    )
    def agg_kernel(hp_hbm, zeros_hbm, src_hbm, dst_hbm, out_hbm,
                   sidx, didx, buf, acc, sem):
        c = lax.axis_index("c")
        s = lax.axis_index("s")
        wid = c * NS + s

        @pl.when(c == 0)
        def _():
            _sliced_copy(s, n,
                         lambda o, r: hp_hbm.at[pl.ds(o, r)],
                         lambda o, r: acc.at[pl.ds(o, r)])

        @pl.when(c != 0)
        def _():
            _sliced_copy(s, n,
                         lambda o, r: zeros_hbm.at[pl.ds(o, r)],
                         lambda o, r: acc.at[pl.ds(o, r)])

        pltpu.sync_copy(src_hbm.at[wid], sidx)
        pltpu.sync_copy(dst_hbm.at[wid], didx)
        plsc.subcore_barrier()

        @pl.loop(0, k)
        def _(j):
            pltpu.async_copy(hp_hbm.at[sidx.at[j]], buf, sem).wait()
            pltpu.sync_copy(buf, acc.at[didx.at[j]], add=True)

        plsc.subcore_barrier()
        _sliced_copy(s, n,
                     lambda o, r: acc.at[pl.ds(o, r)],
                     lambda o, r: out_hbm.at[c].at[pl.ds(o, r)])

    return agg_kernel(hp, zeros_nd, srcr, dstr)


def _matmul_call(x, w):
    n, d = x.shape
    do = w.shape[1]

    def body(x_ref, w_ref, o_ref):
        o_ref[...] = jnp.dot(x_ref[...], w_ref[...],
                             preferred_element_type=jnp.float32)

    return pl.pallas_call(
        body,
        grid=(n // ROWS_TC,),
        in_specs=[pl.BlockSpec((ROWS_TC, d), lambda i: (i, 0)),
                  pl.BlockSpec((d, do), lambda i: (0, 0))],
        out_specs=pl.BlockSpec((ROWS_TC, do), lambda i: (i, 0)),
        out_shape=jax.ShapeDtypeStruct((n, do), jnp.float32),
    )(x, w)


def _dis_block(deg_ref):
    cnt = deg_ref[0, :, 0:1] + deg_ref[1, :, 0:1]
    return lax.rsqrt(cnt + 1.0)


def _scale_call(deg, h):
    n, d = h.shape

    def body(deg_ref, h_ref, o_ref):
        o_ref[...] = h_ref[...] * _dis_block(deg_ref)

    return pl.pallas_call(
        body,
        grid=(n // ROWS_TC,),
        in_specs=[pl.BlockSpec((NC, ROWS_TC, d), lambda i: (0, i, 0)),
                  pl.BlockSpec((ROWS_TC, d), lambda i: (i, 0))],
        out_specs=pl.BlockSpec((ROWS_TC, d), lambda i: (i, 0)),
        out_shape=jax.ShapeDtypeStruct((n, d), jnp.float32),
    )(deg, h)


def _mid_call(acc, deg, b, w):
    _, n, d = acc.shape
    do = w.shape[1]

    def body(acc_ref, deg_ref, b_ref, w_ref, o_ref):
        dis = _dis_block(deg_ref)
        x1 = jnp.maximum((acc_ref[0] + acc_ref[1]) * dis + b_ref[...], 0.0)
        o_ref[...] = jnp.dot(x1, w_ref[...],
                             preferred_element_type=jnp.float32) * dis

    return pl.pallas_call(
        body,
        grid=(n // ROWS_TC,),
        in_specs=[pl.BlockSpec((NC, ROWS_TC, d), lambda i: (0, i, 0)),
                  pl.BlockSpec((NC, ROWS_TC, d), lambda i: (0, i, 0)),
                  pl.BlockSpec((1, d), lambda i: (0, 0)),
                  pl.BlockSpec((d, do), lambda i: (0, 0))],
        out_specs=pl.BlockSpec((ROWS_TC, do), lambda i: (i, 0)),
        out_shape=jax.ShapeDtypeStruct((n, do), jnp.float32),
    )(acc, deg, b, w)


def _final_call(acc, deg, b):
    _, n, d = acc.shape

    def body(acc_ref, deg_ref, b_ref, o_ref):
        dis = _dis_block(deg_ref)
        o_ref[...] = (acc_ref[0] + acc_ref[1]) * dis + b_ref[...]

    return pl.pallas_call(
        body,
        grid=(n // ROWS_TC,),
        in_specs=[pl.BlockSpec((NC, ROWS_TC, d), lambda i: (0, i, 0)),
                  pl.BlockSpec((NC, ROWS_TC, d), lambda i: (0, i, 0)),
                  pl.BlockSpec((1, d), lambda i: (0, 0))],
        out_specs=pl.BlockSpec((ROWS_TC, d), lambda i: (i, 0)),
        out_shape=jax.ShapeDtypeStruct((n, d), jnp.float32),
    )(acc, deg, b)


def kernel(e_prev, edge_index, W1, b1, W2, b2):
    n, _ = e_prev.shape
    d_hid = W1.shape[1]
    d_out = W2.shape[1]
    e = edge_index.shape[1]
    n_pad = n + PAD_ROWS
    k = -(-e // (NW * LW))
    e_pad = NW * k * LW
    pad = e_pad - e

    src = edge_index[0]
    dst = edge_index[1]
    srcr = jnp.concatenate(
        [src, jnp.zeros((pad,), jnp.int32)]).reshape(NW, k, LW)
    dstr = jnp.concatenate(
        [dst, jnp.full((pad,), n, jnp.int32)]).reshape(NW, k, LW)

    zeros_nd = jnp.zeros((n, d_hid), jnp.float32)
    ones_rows = jnp.ones((LW, d_hid), jnp.float32)

    deg = _deg_call(n, n_pad, k, d_hid, dstr, zeros_nd, ones_rows)
    h1 = _matmul_call(e_prev, W1)
    hp1 = _scale_call(deg, h1)
    acc1 = _agg_call(n, n_pad, k, d_hid, hp1, zeros_nd, srcr, dstr)
    hp2 = _mid_call(acc1, deg, b1.reshape(1, d_hid), W2)
    acc2 = _agg_call(n, n_pad, k, d_out, hp2, zeros_nd, srcr, dstr)
    return _final_call(acc2, deg, b2.reshape(1, d_out))


# speedup vs baseline: 15.2110x; 1.2674x over previous
import functools

import jax
import jax.numpy as jnp
from jax import lax
from jax.experimental import pallas as pl
from jax.experimental.pallas import tpu as pltpu
from jax.experimental.pallas import tpu_sc as plsc

NC = 2
NS = 16
NW = NC * NS
LW = 128
IDX_BITS = 14
PAD_ROWS = 16
ROWS_TC = 2000


def _sc_mesh():
    return plsc.VectorSubcoreMesh(core_axis_name="c", subcore_axis_name="s")


CH = 624


def _sliced_copy(s, total, src_at, dst_at):
    last = total - (NS - 1) * CH
    off = pl.multiple_of(s * CH, 8)

    @pl.when(s < NS - 1)
    def _():
        pltpu.sync_copy(src_at(off, CH), dst_at(off, CH))

    @pl.when(s == NS - 1)
    def _():
        o = (NS - 1) * CH
        pltpu.sync_copy(src_at(o, last), dst_at(o, last))


def _deg_call(n, n_pad, k, d, dstr, zeros_nd, ones_rows):
    @functools.partial(
        pl.kernel,
        mesh=_sc_mesh(),
        out_type=jax.ShapeDtypeStruct((NC, n, d), jnp.float32),
        scratch_types=[
            pltpu.VMEM((k, LW), jnp.int32),
            pltpu.VMEM((LW, d), jnp.float32),
            pltpu.VMEM_SHARED((n_pad, d), jnp.float32),
            pltpu.SemaphoreType.DMA,
        ],
    )
    def deg_kernel(dst_hbm, zeros_hbm, ones_hbm, out_hbm, didx, ones_v, acc, sem):
        c = lax.axis_index("c")
        s = lax.axis_index("s")
        wid = c * NS + s
        _sliced_copy(s, n,
                     lambda o, r: zeros_hbm.at[pl.ds(o, r)],
                     lambda o, r: acc.at[pl.ds(o, r)])
        pltpu.sync_copy(ones_hbm, ones_v)
        pltpu.sync_copy(dst_hbm.at[wid], didx)
        plsc.subcore_barrier()

        @pl.loop(0, k)
        def _(j):
            pltpu.sync_copy(ones_v, acc.at[didx.at[j]], add=True)

        plsc.subcore_barrier()
        _sliced_copy(s, n,
                     lambda o, r: acc.at[pl.ds(o, r)],
                     lambda o, r: out_hbm.at[c].at[pl.ds(o, r)])

    return deg_kernel(dstr, zeros_nd, ones_rows)


def _agg_call(n, n_pad, k, d, hp, zeros_nd, pkr):
    @functools.partial(
        pl.kernel,
        mesh=_sc_mesh(),
        out_type=jax.ShapeDtypeStruct((NC, n, d), jnp.float32),
        scratch_types=[
            pltpu.VMEM((k, LW), jnp.int32),
            pltpu.VMEM((LW,), jnp.int32),
            pltpu.VMEM((LW,), jnp.int32),
            pltpu.VMEM((LW,), jnp.int32),
            pltpu.VMEM((LW,), jnp.int32),
            pltpu.VMEM((LW, d), jnp.float32),
            pltpu.VMEM((LW, d), jnp.float32),
            pltpu.VMEM_SHARED((n_pad, d), jnp.float32),
            pltpu.SemaphoreType.DMA,
            pltpu.SemaphoreType.DMA,
        ],
    )
    def agg_kernel(hp_hbm, zeros_hbm, pk_hbm, out_hbm,
                   pk, sb0, db0, sb1, db1, buf0, buf1, acc, sem0, sem1):
        c = lax.axis_index("c")
        s = lax.axis_index("s")
        wid = c * NS + s

        def unpack(j, sb, db):
            @pl.loop(0, LW, step=16)
            def _(q):
                q = pl.multiple_of(q, 8)
                v = pk[j, pl.ds(q, 16)]
                sb[pl.ds(q, 16)] = lax.bitwise_and(v, (1 << IDX_BITS) - 1)
                db[pl.ds(q, 16)] = lax.shift_right_logical(v, IDX_BITS)

        @pl.when(c == 0)
        def _():
            _sliced_copy(s, n,
                         lambda o, r: hp_hbm.at[pl.ds(o, r)],
                         lambda o, r: acc.at[pl.ds(o, r)])

        @pl.when(c != 0)
        def _():
            _sliced_copy(s, n,
                         lambda o, r: zeros_hbm.at[pl.ds(o, r)],
                         lambda o, r: acc.at[pl.ds(o, r)])

        pltpu.sync_copy(pk_hbm.at[wid], pk)
        plsc.subcore_barrier()

        unpack(0, sb0, db0)
        pltpu.async_copy(hp_hbm.at[sb0], buf0, sem0)
        if k > 1:
            unpack(1, sb1, db1)
            pltpu.async_copy(hp_hbm.at[sb1], buf1, sem1)

        @pl.loop(0, k, step=2)
        def _(j):
            pltpu.make_async_copy(hp_hbm.at[sb0], buf0, sem0).wait()
            pltpu.sync_copy(buf0, acc.at[db0], add=True)

            @pl.when(j + 2 < k)
            def _():
                unpack(j + 2, sb0, db0)
                pltpu.async_copy(hp_hbm.at[sb0], buf0, sem0)

            @pl.when(j + 1 < k)
            def _():
                pltpu.make_async_copy(hp_hbm.at[sb1], buf1, sem1).wait()
                pltpu.sync_copy(buf1, acc.at[db1], add=True)

                @pl.when(j + 3 < k)
                def _():
                    unpack(j + 3, sb1, db1)
                    pltpu.async_copy(hp_hbm.at[sb1], buf1, sem1)

        plsc.subcore_barrier()
        _sliced_copy(s, n,
                     lambda o, r: acc.at[pl.ds(o, r)],
                     lambda o, r: out_hbm.at[c].at[pl.ds(o, r)])

    return agg_kernel(hp, zeros_nd, pkr)


def _matmul_call(x, w):
    n, d = x.shape
    do = w.shape[1]

    def body(x_ref, w_ref, o_ref):
        o_ref[...] = jnp.dot(x_ref[...], w_ref[...],
                             preferred_element_type=jnp.float32)

    return pl.pallas_call(
        body,
        grid=(n // ROWS_TC,),
        in_specs=[pl.BlockSpec((ROWS_TC, d), lambda i: (i, 0)),
                  pl.BlockSpec((d, do), lambda i: (0, 0))],
        out_specs=pl.BlockSpec((ROWS_TC, do), lambda i: (i, 0)),
        out_shape=jax.ShapeDtypeStruct((n, do), jnp.float32),
    )(x, w)


def _dis_block(deg_ref):
    cnt = deg_ref[0, :, 0:1] + deg_ref[1, :, 0:1]
    return lax.rsqrt(cnt + 1.0)


def _scale_call(deg, h):
    n, d = h.shape

    def body(deg_ref, h_ref, o_ref):
        o_ref[...] = h_ref[...] * _dis_block(deg_ref)

    return pl.pallas_call(
        body,
        grid=(n // ROWS_TC,),
        in_specs=[pl.BlockSpec((NC, ROWS_TC, d), lambda i: (0, i, 0)),
                  pl.BlockSpec((ROWS_TC, d), lambda i: (i, 0))],
        out_specs=pl.BlockSpec((ROWS_TC, d), lambda i: (i, 0)),
        out_shape=jax.ShapeDtypeStruct((n, d), jnp.float32),
    )(deg, h)


def _mid_call(acc, deg, b, w):
    _, n, d = acc.shape
    do = w.shape[1]

    def body(acc_ref, deg_ref, b_ref, w_ref, o_ref):
        dis = _dis_block(deg_ref)
        x1 = jnp.maximum((acc_ref[0] + acc_ref[1]) * dis + b_ref[...], 0.0)
        o_ref[...] = jnp.dot(x1, w_ref[...],
                             preferred_element_type=jnp.float32) * dis

    return pl.pallas_call(
        body,
        grid=(n // ROWS_TC,),
        in_specs=[pl.BlockSpec((NC, ROWS_TC, d), lambda i: (0, i, 0)),
                  pl.BlockSpec((NC, ROWS_TC, d), lambda i: (0, i, 0)),
                  pl.BlockSpec((1, d), lambda i: (0, 0)),
                  pl.BlockSpec((d, do), lambda i: (0, 0))],
        out_specs=pl.BlockSpec((ROWS_TC, do), lambda i: (i, 0)),
        out_shape=jax.ShapeDtypeStruct((n, do), jnp.float32),
    )(acc, deg, b, w)


def _final_call(acc, deg, b):
    _, n, d = acc.shape

    def body(acc_ref, deg_ref, b_ref, o_ref):
        dis = _dis_block(deg_ref)
        o_ref[...] = (acc_ref[0] + acc_ref[1]) * dis + b_ref[...]

    return pl.pallas_call(
        body,
        grid=(n // ROWS_TC,),
        in_specs=[pl.BlockSpec((NC, ROWS_TC, d), lambda i: (0, i, 0)),
                  pl.BlockSpec((NC, ROWS_TC, d), lambda i: (0, i, 0)),
                  pl.BlockSpec((1, d), lambda i: (0, 0))],
        out_specs=pl.BlockSpec((ROWS_TC, d), lambda i: (i, 0)),
        out_shape=jax.ShapeDtypeStruct((n, d), jnp.float32),
    )(acc, deg, b)


def kernel(e_prev, edge_index, W1, b1, W2, b2):
    n, _ = e_prev.shape
    d_hid = W1.shape[1]
    d_out = W2.shape[1]
    e = edge_index.shape[1]
    n_pad = n + PAD_ROWS
    k = -(-e // (NW * LW))
    e_pad = NW * k * LW
    pad = e_pad - e

    src = edge_index[0]
    dst = edge_index[1]
    srcp = jnp.concatenate([src, jnp.zeros((pad,), jnp.int32)])
    dstp = jnp.concatenate([dst, jnp.full((pad,), n, jnp.int32)])
    dstr = dstp.reshape(NW, k, LW)
    pkr = (srcp | (dstp << IDX_BITS)).reshape(NW, k, LW)

    zeros_nd = jnp.zeros((n, d_hid), jnp.float32)
    ones_rows = jnp.ones((LW, d_hid), jnp.float32)

    deg = _deg_call(n, n_pad, k, d_hid, dstr, zeros_nd, ones_rows)
    h1 = _matmul_call(e_prev, W1)
    hp1 = _scale_call(deg, h1)
    acc1 = _agg_call(n, n_pad, k, d_hid, hp1, zeros_nd, pkr)
    hp2 = _mid_call(acc1, deg, b1.reshape(1, d_hid), W2)
    acc2 = _agg_call(n, n_pad, k, d_out, hp2, zeros_nd, pkr)
    return _final_call(acc2, deg, b2.reshape(1, d_out))


# speedup vs baseline: 16.1540x; 1.0620x over previous
import functools

import jax
import jax.numpy as jnp
from jax import lax
from jax.experimental import pallas as pl
from jax.experimental.pallas import tpu as pltpu
from jax.experimental.pallas import tpu_sc as plsc

NC = 2
NS = 16
NW = NC * NS
LW = 128
IDX_BITS = 14
PAD_ROWS = 16
ROWS_TC = 2000


def _sc_mesh():
    return plsc.VectorSubcoreMesh(core_axis_name="c", subcore_axis_name="s")


CH = 624


def _sliced_copy(s, total, src_at, dst_at):
    last = total - (NS - 1) * CH
    off = pl.multiple_of(s * CH, 8)

    @pl.when(s < NS - 1)
    def _():
        pltpu.sync_copy(src_at(off, CH), dst_at(off, CH))

    @pl.when(s == NS - 1)
    def _():
        o = (NS - 1) * CH
        pltpu.sync_copy(src_at(o, last), dst_at(o, last))


def _deg_call(n, n_pad, k, d, dstr, zeros_nd, ones_rows):
    @functools.partial(
        pl.kernel,
        mesh=_sc_mesh(),
        out_type=jax.ShapeDtypeStruct((NC, n, d), jnp.float32),
        scratch_types=[
            pltpu.VMEM((k, LW), jnp.int32),
            pltpu.VMEM((LW, d), jnp.float32),
            pltpu.VMEM_SHARED((n_pad, d), jnp.float32),
            pltpu.SemaphoreType.DMA,
        ],
    )
    def deg_kernel(dst_hbm, zeros_hbm, ones_hbm, out_hbm, didx, ones_v, acc, sem):
        c = lax.axis_index("c")
        s = lax.axis_index("s")
        wid = c * NS + s
        _sliced_copy(s, n,
                     lambda o, r: zeros_hbm.at[pl.ds(o, r)],
                     lambda o, r: acc.at[pl.ds(o, r)])
        pltpu.sync_copy(ones_hbm, ones_v)
        pltpu.sync_copy(dst_hbm.at[wid], didx)
        plsc.subcore_barrier()

        @pl.loop(0, k)
        def _(j):
            pltpu.sync_copy(ones_v, acc.at[didx.at[j]], add=True)

        plsc.subcore_barrier()
        _sliced_copy(s, n,
                     lambda o, r: acc.at[pl.ds(o, r)],
                     lambda o, r: out_hbm.at[c].at[pl.ds(o, r)])

    return deg_kernel(dstr, zeros_nd, ones_rows)


def _agg_call(n, n_pad, k, d, hp, zeros_nd, pkr):
    @functools.partial(
        pl.kernel,
        mesh=_sc_mesh(),
        out_type=jax.ShapeDtypeStruct((NC, n, d), jnp.float32),
        scratch_types=[
            pltpu.VMEM((k, LW), jnp.int32),
            pltpu.VMEM((LW,), jnp.int32),
            pltpu.VMEM((LW,), jnp.int32),
            pltpu.VMEM((LW,), jnp.int32),
            pltpu.VMEM((LW,), jnp.int32),
            pltpu.VMEM((LW, d), jnp.float32),
            pltpu.VMEM((LW, d), jnp.float32),
            pltpu.VMEM_SHARED((n_pad, d), jnp.float32),
            pltpu.SemaphoreType.DMA,
            pltpu.SemaphoreType.DMA,
        ],
    )
    def agg_kernel(hp_hbm, zeros_hbm, pk_hbm, out_hbm,
                   pk, sb0, db0, sb1, db1, buf0, buf1, acc, sem0, sem1):
        c = lax.axis_index("c")
        s = lax.axis_index("s")
        wid = c * NS + s

        def unpack(j, sb, db):
            @pl.loop(0, LW, step=16)
            def _(q):
                q = pl.multiple_of(q, 8)
                v = pk[j, pl.ds(q, 16)]
                sb[pl.ds(q, 16)] = lax.bitwise_and(v, (1 << IDX_BITS) - 1)
                db[pl.ds(q, 16)] = lax.shift_right_logical(v, IDX_BITS)

        @pl.when(c == 0)
        def _():
            _sliced_copy(s, n,
                         lambda o, r: hp_hbm.at[pl.ds(o, r)],
                         lambda o, r: acc.at[pl.ds(o, r)])

        @pl.when(c != 0)
        def _():
            _sliced_copy(s, n,
                         lambda o, r: zeros_hbm.at[pl.ds(o, r)],
                         lambda o, r: acc.at[pl.ds(o, r)])

        pltpu.sync_copy(pk_hbm.at[wid], pk)
        plsc.subcore_barrier()

        unpack(0, sb0, db0)
        pltpu.async_copy(hp_hbm.at[sb0], buf0, sem0)
        if k > 1:
            unpack(1, sb1, db1)
            pltpu.async_copy(hp_hbm.at[sb1], buf1, sem1)

        @pl.loop(0, k, step=2)
        def _(j):
            pltpu.make_async_copy(hp_hbm.at[sb0], buf0, sem0).wait()
            pltpu.sync_copy(buf0, acc.at[db0], add=True)

            @pl.when(j + 2 < k)
            def _():
                unpack(j + 2, sb0, db0)
                pltpu.async_copy(hp_hbm.at[sb0], buf0, sem0)

            @pl.when(j + 1 < k)
            def _():
                pltpu.make_async_copy(hp_hbm.at[sb1], buf1, sem1).wait()
                pltpu.sync_copy(buf1, acc.at[db1], add=True)

                @pl.when(j + 3 < k)
                def _():
                    unpack(j + 3, sb1, db1)
                    pltpu.async_copy(hp_hbm.at[sb1], buf1, sem1)

        plsc.subcore_barrier()
        _sliced_copy(s, n,
                     lambda o, r: acc.at[pl.ds(o, r)],
                     lambda o, r: out_hbm.at[c].at[pl.ds(o, r)])

    return agg_kernel(hp, zeros_nd, pkr)


def _matmul_call(x, w):
    n, d = x.shape
    do = w.shape[1]

    def body(x_ref, w_ref, o_ref):
        o_ref[...] = jnp.dot(x_ref[...], w_ref[...],
                             preferred_element_type=jnp.float32)

    return pl.pallas_call(
        body,
        grid=(n // ROWS_TC,),
        in_specs=[pl.BlockSpec((ROWS_TC, d), lambda i: (i, 0)),
                  pl.BlockSpec((d, do), lambda i: (0, 0))],
        out_specs=pl.BlockSpec((ROWS_TC, do), lambda i: (i, 0)),
        out_shape=jax.ShapeDtypeStruct((n, do), jnp.float32),
    )(x, w)


def _dis_block(deg_ref):
    cnt = deg_ref[0, :, 0:1] + deg_ref[1, :, 0:1]
    return lax.rsqrt(cnt + 1.0)


def _scale_call(deg, h):
    n, d = h.shape

    def body(deg_ref, h_ref, o_ref):
        o_ref[...] = h_ref[...] * _dis_block(deg_ref)

    return pl.pallas_call(
        body,
        grid=(n // ROWS_TC,),
        in_specs=[pl.BlockSpec((NC, ROWS_TC, d), lambda i: (0, i, 0)),
                  pl.BlockSpec((ROWS_TC, d), lambda i: (i, 0))],
        out_specs=pl.BlockSpec((ROWS_TC, d), lambda i: (i, 0)),
        out_shape=jax.ShapeDtypeStruct((n, d), jnp.float32),
    )(deg, h)


def _mid_call(acc, deg, b, w):
    _, n, d = acc.shape
    do = w.shape[1]

    def body(acc_ref, deg_ref, b_ref, w_ref, o_ref):
        dis = _dis_block(deg_ref)
        x1 = jnp.maximum((acc_ref[0] + acc_ref[1]) * dis + b_ref[...], 0.0)
        o_ref[...] = jnp.dot(x1, w_ref[...],
                             preferred_element_type=jnp.float32) * dis

    return pl.pallas_call(
        body,
        grid=(n // ROWS_TC,),
        in_specs=[pl.BlockSpec((NC, ROWS_TC, d), lambda i: (0, i, 0)),
                  pl.BlockSpec((NC, ROWS_TC, d), lambda i: (0, i, 0)),
                  pl.BlockSpec((1, d), lambda i: (0, 0)),
                  pl.BlockSpec((d, do), lambda i: (0, 0))],
        out_specs=pl.BlockSpec((ROWS_TC, do), lambda i: (i, 0)),
        out_shape=jax.ShapeDtypeStruct((n, do), jnp.float32),
    )(acc, deg, b, w)


def _final_call(acc, deg, b):
    _, n, d = acc.shape

    def body(acc_ref, deg_ref, b_ref, o_ref):
        dis = _dis_block(deg_ref)
        o_ref[...] = (acc_ref[0] + acc_ref[1]) * dis + b_ref[...]

    return pl.pallas_call(
        body,
        grid=(n // ROWS_TC,),
        in_specs=[pl.BlockSpec((NC, ROWS_TC, d), lambda i: (0, i, 0)),
                  pl.BlockSpec((NC, ROWS_TC, d), lambda i: (0, i, 0)),
                  pl.BlockSpec((1, d), lambda i: (0, 0))],
        out_specs=pl.BlockSpec((ROWS_TC, d), lambda i: (i, 0)),
        out_shape=jax.ShapeDtypeStruct((n, d), jnp.float32),
    )(acc, deg, b)


def kernel(e_prev, edge_index, W1, b1, W2, b2):
    n, _ = e_prev.shape
    d_hid = W1.shape[1]
    d_out = W2.shape[1]
    e = edge_index.shape[1]
    n_pad = n + PAD_ROWS
    k = -(-e // (NW * LW))
    e_pad = NW * k * LW
    pad = e_pad - e

    src = edge_index[0]
    dst = edge_index[1]
    srcp = jnp.concatenate([src, jnp.zeros((pad,), jnp.int32)])
    dstp = jnp.concatenate([dst, jnp.full((pad,), n, jnp.int32)])
    dstr = dstp.reshape(k, NW, LW).transpose(1, 0, 2)
    pkr = (srcp | (dstp << IDX_BITS)).reshape(k, NW, LW).transpose(1, 0, 2)

    zeros_nd = jnp.zeros((n, d_hid), jnp.float32)
    ones_rows = jnp.ones((LW, d_hid), jnp.float32)

    deg = _deg_call(n, n_pad, k, d_hid, dstr, zeros_nd, ones_rows)
    h1 = _matmul_call(e_prev, W1)
    hp1 = _scale_call(deg, h1)
    acc1 = _agg_call(n, n_pad, k, d_hid, hp1, zeros_nd, pkr)
    hp2 = _mid_call(acc1, deg, b1.reshape(1, d_hid), W2)
    acc2 = _agg_call(n, n_pad, k, d_out, hp2, zeros_nd, pkr)
    return _final_call(acc2, deg, b2.reshape(1, d_out))


# speedup vs baseline: 16.1917x; 1.0023x over previous
import functools

import jax
import jax.numpy as jnp
from jax import lax
from jax.experimental import pallas as pl
from jax.experimental.pallas import tpu as pltpu
from jax.experimental.pallas import tpu_sc as plsc

NC = 2
NS = 16
NW = NC * NS
LW = 128
IDX_BITS = 14
PAD_ROWS = 128
ROWS_TC = 2000


def _sc_mesh():
    return plsc.VectorSubcoreMesh(core_axis_name="c", subcore_axis_name="s")


CH = 624


def _sliced_copy(s, total, src_at, dst_at):
    last = total - (NS - 1) * CH
    off = pl.multiple_of(s * CH, 8)

    @pl.when(s < NS - 1)
    def _():
        pltpu.sync_copy(src_at(off, CH), dst_at(off, CH))

    @pl.when(s == NS - 1)
    def _():
        o = (NS - 1) * CH
        pltpu.sync_copy(src_at(o, last), dst_at(o, last))


def _deg_call(n, n_pad, k, d, dstr, zeros_nd, ones_rows):
    @functools.partial(
        pl.kernel,
        mesh=_sc_mesh(),
        out_type=jax.ShapeDtypeStruct((NC, n, d), jnp.float32),
        scratch_types=[
            pltpu.VMEM((k, LW), jnp.int32),
            pltpu.VMEM((LW, d), jnp.float32),
            pltpu.VMEM_SHARED((n_pad, d), jnp.float32),
            pltpu.SemaphoreType.DMA,
        ],
    )
    def deg_kernel(dst_hbm, zeros_hbm, ones_hbm, out_hbm, didx, ones_v, acc, sem):
        c = lax.axis_index("c")
        s = lax.axis_index("s")
        wid = c * NS + s
        _sliced_copy(s, n,
                     lambda o, r: zeros_hbm.at[pl.ds(o, r)],
                     lambda o, r: acc.at[pl.ds(o, r)])
        pltpu.sync_copy(ones_hbm, ones_v)
        pltpu.sync_copy(dst_hbm.at[wid], didx)
        plsc.subcore_barrier()

        @pl.loop(0, k)
        def _(j):
            pltpu.sync_copy(ones_v, acc.at[didx.at[j]], add=True)

        plsc.subcore_barrier()
        _sliced_copy(s, n,
                     lambda o, r: acc.at[pl.ds(o, r)],
                     lambda o, r: out_hbm.at[c].at[pl.ds(o, r)])

    return deg_kernel(dstr, zeros_nd, ones_rows)


def _agg_call(n, n_pad, k, d, hp, zeros_nd, pkr):
    @functools.partial(
        pl.kernel,
        mesh=_sc_mesh(),
        out_type=jax.ShapeDtypeStruct((NC, n, d), jnp.float32),
        scratch_types=[
            pltpu.VMEM((k, LW), jnp.int32),
            pltpu.VMEM((LW,), jnp.int32),
            pltpu.VMEM((LW,), jnp.int32),
            pltpu.VMEM((LW,), jnp.int32),
            pltpu.VMEM((LW,), jnp.int32),
            pltpu.VMEM((LW, d), jnp.float32),
            pltpu.VMEM((LW, d), jnp.float32),
            pltpu.VMEM_SHARED((n_pad, d), jnp.float32),
            pltpu.SemaphoreType.DMA,
            pltpu.SemaphoreType.DMA,
        ],
    )
    def agg_kernel(hp_hbm, zeros_hbm, pk_hbm, out_hbm,
                   pk, sb0, db0, sb1, db1, buf0, buf1, acc, sem0, sem1):
        c = lax.axis_index("c")
        s = lax.axis_index("s")
        wid = c * NS + s

        def unpack(j, sb, db):
            @pl.loop(0, LW, step=16)
            def _(q):
                q = pl.multiple_of(q, 8)
                v = pk[j, pl.ds(q, 16)]
                sb[pl.ds(q, 16)] = lax.bitwise_and(v, (1 << IDX_BITS) - 1)
                db[pl.ds(q, 16)] = lax.shift_right_logical(v, IDX_BITS)

        @pl.when(c == 0)
        def _():
            _sliced_copy(s, n,
                         lambda o, r: hp_hbm.at[pl.ds(o, r)],
                         lambda o, r: acc.at[pl.ds(o, r)])

        @pl.when(c != 0)
        def _():
            _sliced_copy(s, n,
                         lambda o, r: zeros_hbm.at[pl.ds(o, r)],
                         lambda o, r: acc.at[pl.ds(o, r)])

        pltpu.sync_copy(pk_hbm.at[wid], pk)
        plsc.subcore_barrier()

        unpack(0, sb0, db0)
        pltpu.async_copy(hp_hbm.at[sb0], buf0, sem0)
        if k > 1:
            unpack(1, sb1, db1)
            pltpu.async_copy(hp_hbm.at[sb1], buf1, sem1)

        @pl.loop(0, k, step=2)
        def _(j):
            pltpu.make_async_copy(hp_hbm.at[sb0], buf0, sem0).wait()
            pltpu.sync_copy(buf0, acc.at[db0], add=True)

            @pl.when(j + 2 < k)
            def _():
                unpack(j + 2, sb0, db0)
                pltpu.async_copy(hp_hbm.at[sb0], buf0, sem0)

            @pl.when(j + 1 < k)
            def _():
                pltpu.make_async_copy(hp_hbm.at[sb1], buf1, sem1).wait()
                pltpu.sync_copy(buf1, acc.at[db1], add=True)

                @pl.when(j + 3 < k)
                def _():
                    unpack(j + 3, sb1, db1)
                    pltpu.async_copy(hp_hbm.at[sb1], buf1, sem1)

        plsc.subcore_barrier()
        _sliced_copy(s, n,
                     lambda o, r: acc.at[pl.ds(o, r)],
                     lambda o, r: out_hbm.at[c].at[pl.ds(o, r)])

    return agg_kernel(hp, zeros_nd, pkr)


def _matmul_call(x, w):
    n, d = x.shape
    do = w.shape[1]

    def body(x_ref, w_ref, o_ref):
        o_ref[...] = jnp.dot(x_ref[...], w_ref[...],
                             preferred_element_type=jnp.float32)

    return pl.pallas_call(
        body,
        grid=(n // ROWS_TC,),
        in_specs=[pl.BlockSpec((ROWS_TC, d), lambda i: (i, 0)),
                  pl.BlockSpec((d, do), lambda i: (0, 0))],
        out_specs=pl.BlockSpec((ROWS_TC, do), lambda i: (i, 0)),
        out_shape=jax.ShapeDtypeStruct((n, do), jnp.float32),
    )(x, w)


def _dis_block(deg_ref):
    cnt = deg_ref[0, :, 0:1] + deg_ref[1, :, 0:1]
    return lax.rsqrt(cnt + 1.0)


def _scale_call(deg, h):
    n, d = h.shape

    def body(deg_ref, h_ref, o_ref):
        o_ref[...] = h_ref[...] * _dis_block(deg_ref)

    return pl.pallas_call(
        body,
        grid=(n // ROWS_TC,),
        in_specs=[pl.BlockSpec((NC, ROWS_TC, d), lambda i: (0, i, 0)),
                  pl.BlockSpec((ROWS_TC, d), lambda i: (i, 0))],
        out_specs=pl.BlockSpec((ROWS_TC, d), lambda i: (i, 0)),
        out_shape=jax.ShapeDtypeStruct((n, d), jnp.float32),
    )(deg, h)


def _mid_call(acc, deg, b, w):
    _, n, d = acc.shape
    do = w.shape[1]

    def body(acc_ref, deg_ref, b_ref, w_ref, o_ref):
        dis = _dis_block(deg_ref)
        x1 = jnp.maximum((acc_ref[0] + acc_ref[1]) * dis + b_ref[...], 0.0)
        o_ref[...] = jnp.dot(x1, w_ref[...],
                             preferred_element_type=jnp.float32) * dis

    return pl.pallas_call(
        body,
        grid=(n // ROWS_TC,),
        in_specs=[pl.BlockSpec((NC, ROWS_TC, d), lambda i: (0, i, 0)),
                  pl.BlockSpec((NC, ROWS_TC, d), lambda i: (0, i, 0)),
                  pl.BlockSpec((1, d), lambda i: (0, 0)),
                  pl.BlockSpec((d, do), lambda i: (0, 0))],
        out_specs=pl.BlockSpec((ROWS_TC, do), lambda i: (i, 0)),
        out_shape=jax.ShapeDtypeStruct((n, do), jnp.float32),
    )(acc, deg, b, w)


def _final_call(acc, deg, b):
    _, n, d = acc.shape

    def body(acc_ref, deg_ref, b_ref, o_ref):
        dis = _dis_block(deg_ref)
        o_ref[...] = (acc_ref[0] + acc_ref[1]) * dis + b_ref[...]

    return pl.pallas_call(
        body,
        grid=(n // ROWS_TC,),
        in_specs=[pl.BlockSpec((NC, ROWS_TC, d), lambda i: (0, i, 0)),
                  pl.BlockSpec((NC, ROWS_TC, d), lambda i: (0, i, 0)),
                  pl.BlockSpec((1, d), lambda i: (0, 0))],
        out_specs=pl.BlockSpec((ROWS_TC, d), lambda i: (i, 0)),
        out_shape=jax.ShapeDtypeStruct((n, d), jnp.float32),
    )(acc, deg, b)


def kernel(e_prev, edge_index, W1, b1, W2, b2):
    n, _ = e_prev.shape
    d_hid = W1.shape[1]
    d_out = W2.shape[1]
    e = edge_index.shape[1]
    n_pad = n + PAD_ROWS
    k = -(-e // (NW * LW))
    e_pad = NW * k * LW
    pad = e_pad - e

    src = edge_index[0]
    dst = edge_index[1]
    srcp = jnp.concatenate([src, jnp.zeros((pad,), jnp.int32)])
    dstp = jnp.concatenate(
        [dst, n + (jnp.arange(pad, dtype=jnp.int32) % PAD_ROWS)])
    dstr = dstp.reshape(k, NW, LW).transpose(1, 0, 2)
    pkr = (srcp | (dstp << IDX_BITS)).reshape(k, NW, LW).transpose(1, 0, 2)

    zeros_nd = jnp.zeros((n, d_hid), jnp.float32)
    ones_rows = jnp.ones((LW, d_hid), jnp.float32)

    deg = _deg_call(n, n_pad, k, d_hid, dstr, zeros_nd, ones_rows)
    h1 = _matmul_call(e_prev, W1)
    hp1 = _scale_call(deg, h1)
    acc1 = _agg_call(n, n_pad, k, d_hid, hp1, zeros_nd, pkr)
    hp2 = _mid_call(acc1, deg, b1.reshape(1, d_hid), W2)
    acc2 = _agg_call(n, n_pad, k, d_out, hp2, zeros_nd, pkr)
    return _final_call(acc2, deg, b2.reshape(1, d_out))
